# baseline (speedup 1.0000x reference)
import jax
import jax.numpy as jnp
from jax.experimental import pallas as pl
from jax.experimental.pallas import tpu as pltpu
from jax.experimental.pallas import tpu_sc as plsc

N = 4096
D = 128
NHID = 128
NCLASS = 128
ALPHA = 0.2
R = 256
GW = 128
NBLK = N // R
LOG2E = 1.4426950408889634


def _sc_gather(table, idx):
    n = idx.shape[0]
    d = table.shape[1]
    idx2 = idx.reshape(1, n)
    mesh = plsc.VectorSubcoreMesh(core_axis_name="core",
                                  subcore_axis_name="subcore")

    @pl.kernel(out_type=jax.ShapeDtypeStruct((n, d), table.dtype), mesh=mesh)
    def gather_kernel(tab_hbm, i_hbm, o_hbm):
        def body(i_vmem, o_vmem):
            pltpu.sync_copy(tab_hbm.at[i_vmem.at[0]], o_vmem)

        pltpu.emit_pipeline(
            body,
            grid=(n // GW,),
            in_specs=[pl.BlockSpec((1, GW), index_map=lambda i: (0, i))],
            out_specs=[pl.BlockSpec((GW, d), index_map=lambda i: (i, 0))],
            core_axis_name=("core", "subcore"),
            dimension_semantics=(pltpu.PARALLEL,),
        )(i_hbm, o_hbm)

    return gather_kernel(table, idx2)


def _leaky(x):
    return jnp.maximum(x, ALPHA * x)


def _elu(x):
    return jnp.where(x > 0, x, jnp.exp(x) - 1.0)


def _logit_vecs(wh, a_row):
    a1 = a_row[:, :D] * LOG2E
    a2 = a_row[:, D:] * LOG2E
    f1 = jnp.sum(wh * a1, axis=1, keepdims=True)
    f2 = jax.lax.dot_general(a2, wh, (((1,), (1,)), ((), ())),
                             preferred_element_type=jnp.float32)
    return f1, f2


def _att_block(adjv, f1, f2, w_bf, cm):
    bf16 = jnp.bfloat16
    m = _leaky(f1 + jnp.max(f2))
    u_row = (f1 - m).astype(bf16)
    v_row = (ALPHA * f1 - m).astype(bf16)
    f2_bf = f2.astype(bf16)
    f2s = (ALPHA * f2).astype(bf16)
    CN = 1024
    hpe = jnp.zeros((adjv.shape[0], D + 128), jnp.float32)
    for k in range(N // CN):
        lo, hi = k * CN, (k + 1) * CN
        x = jnp.maximum(u_row + f2_bf[:, lo:hi], v_row + f2s[:, lo:hi])
        p = adjv[:, lo:hi] * jnp.exp2(x)
        hpe = hpe + jnp.dot(p, w_bf[lo:hi, :],
                            preferred_element_type=jnp.float32)
    hp = hpe[:, :D]
    s = hpe[:, D:D + 1]
    safe = jnp.where(s > 0, s, 1.0)
    return jnp.where(s > 0, hp / safe, cm)


def _with_ones_col(w_bf):
    rows = w_bf.shape[0]
    lane = jax.lax.broadcasted_iota(jnp.int32, (rows, D), 1)
    ones_blk = jnp.where(lane == 0, 1.0, 0.0).astype(jnp.bfloat16)
    return jnp.concatenate([w_bf, ones_blk], axis=1)


def _mega_body(x_ref, adj_ref, w1a_ref, w1b_ref, a1a_ref, a1b_ref,
               wout_ref, aout_ref, out_ref,
               wa_ref, wb_ref, f1a_ref, f2a_ref, f1b_ref, f2b_ref,
               cma_ref, cmb_ref,
               who_ref, f1o_ref, f2o_ref, cmo_ref):
    i = pl.program_id(0)

    @pl.when(i == 0)
    def _prewh():
        x = x_ref[...]
        wa = jnp.dot(x, w1a_ref[...], preferred_element_type=jnp.float32)
        wb = jnp.dot(x, w1b_ref[...], preferred_element_type=jnp.float32)
        wa_ref[...] = _with_ones_col(wa.astype(jnp.bfloat16))
        wb_ref[...] = _with_ones_col(wb.astype(jnp.bfloat16))
        f1a_ref[...], f2a_ref[...] = _logit_vecs(wa, a1a_ref[...])
        f1b_ref[...], f2b_ref[...] = _logit_vecs(wb, a1b_ref[...])
        cma_ref[...] = jnp.mean(wa, axis=0, keepdims=True)
        cmb_ref[...] = jnp.mean(wb, axis=0, keepdims=True)
        cmo_ref[...] = jnp.zeros_like(cmo_ref)

    @pl.when((i >= 1) & (i <= NBLK))
    def _layer1():
        r0 = (i - 1) * R
        adjv = adj_ref[...].astype(jnp.bfloat16)
        ha = _elu(_att_block(adjv, f1a_ref[pl.ds(r0, R), :], f2a_ref[...],
                             wa_ref[...], cma_ref[...]))
        hb = _elu(_att_block(adjv, f1b_ref[pl.ds(r0, R), :], f2b_ref[...],
                             wb_ref[...], cmb_ref[...]))
        h1 = jnp.concatenate([ha, hb], axis=1)
        who = jnp.dot(h1, wout_ref[...], preferred_element_type=jnp.float32)
        who_ref[pl.ds(r0, R), :] = _with_ones_col(who.astype(jnp.bfloat16))
        f1o, f2o = _logit_vecs(who, aout_ref[...])
        f1o_ref[pl.ds(r0, R), :] = f1o
        f2o_ref[:, pl.ds(r0, R)] = f2o
        cmo_ref[...] += jnp.sum(who, axis=0, keepdims=True) * (1.0 / N)

    @pl.when(i > NBLK)
    def _layer2():
        r0 = (i - NBLK - 1) * R
        out = _elu(_att_block(adj_ref[...].astype(jnp.bfloat16),
                              f1o_ref[pl.ds(r0, R), :],
                              f2o_ref[...], who_ref[...], cmo_ref[...]))
        m = jnp.max(out, axis=1, keepdims=True)
        lse = jnp.log(jnp.sum(jnp.exp(out - m), axis=1, keepdims=True))
        out_ref[...] = out - m - lse


def _full(shape):
    return pl.BlockSpec(shape, lambda i: (0, 0))


def kernel(idx, adj, table, W1a, a1a, W1b, a1b, Wout, aout):
    features = _sc_gather(table, idx)

    f32 = jnp.float32
    vmem = pltpu.VMEM
    out = pl.pallas_call(
        _mega_body,
        grid=(2 * NBLK + 1,),
        in_specs=[_full((N, D)),
                  pl.BlockSpec((R, N), lambda i: ((i + NBLK - 1) % NBLK, 0)),
                  _full((D, NHID)), _full((D, NHID)),
                  _full((1, 2 * NHID)), _full((1, 2 * NHID)),
                  _full((2 * NHID, NCLASS)), _full((1, 2 * NCLASS))],
        out_specs=pl.BlockSpec(
            (R, NCLASS), lambda i: (jnp.maximum(i - NBLK - 1, 0), 0)),
        out_shape=jax.ShapeDtypeStruct((N, NCLASS), f32),
        scratch_shapes=[
            vmem((N, NHID + 128), jnp.bfloat16),
            vmem((N, NHID + 128), jnp.bfloat16),
            vmem((N, 1), f32), vmem((1, N), f32),
            vmem((N, 1), f32), vmem((1, N), f32),
            vmem((1, NHID), f32), vmem((1, NHID), f32),
            vmem((N, NCLASS + 128), jnp.bfloat16),
            vmem((N, 1), f32), vmem((1, N), f32),
            vmem((1, NCLASS), f32),
        ],
    )(features, adj, W1a, W1b, a1a.reshape(1, -1), a1b.reshape(1, -1),
      Wout, aout.reshape(1, -1))

    return out


# speedup vs baseline: 1.1360x; 1.1360x over previous
import jax
import jax.numpy as jnp
from jax.experimental import pallas as pl
from jax.experimental.pallas import tpu as pltpu
from jax.experimental.pallas import tpu_sc as plsc

N = 4096
D = 128
NHID = 128
NCLASS = 128
ALPHA = 0.2
R = 512
GW = 128
NBLK = N // R
LOG2E = 1.4426950408889634


def _sc_gather(table, idx):
    n = idx.shape[0]
    d = table.shape[1]
    idx2 = idx.reshape(1, n)
    mesh = plsc.VectorSubcoreMesh(core_axis_name="core",
                                  subcore_axis_name="subcore")

    @pl.kernel(out_type=jax.ShapeDtypeStruct((n, d), table.dtype), mesh=mesh)
    def gather_kernel(tab_hbm, i_hbm, o_hbm):
        def body(i_vmem, o_vmem):
            pltpu.sync_copy(tab_hbm.at[i_vmem.at[0]], o_vmem)

        pltpu.emit_pipeline(
            body,
            grid=(n // GW,),
            in_specs=[pl.BlockSpec((1, GW), index_map=lambda i: (0, i))],
            out_specs=[pl.BlockSpec((GW, d), index_map=lambda i: (i, 0))],
            core_axis_name=("core", "subcore"),
            dimension_semantics=(pltpu.PARALLEL,),
        )(i_hbm, o_hbm)

    return gather_kernel(table, idx2)


def _leaky(x):
    return jnp.maximum(x, ALPHA * x)


def _elu(x):
    return jnp.where(x > 0, x, jnp.exp(x) - 1.0)


def _logit_vecs(wh, a_row):
    a1 = a_row[:, :D] * LOG2E
    a2 = a_row[:, D:] * LOG2E
    f1 = jnp.sum(wh * a1, axis=1, keepdims=True)
    f2 = jax.lax.dot_general(a2, wh, (((1,), (1,)), ((), ())),
                             preferred_element_type=jnp.float32)
    return f1, f2


def _att_block(adjv, f1, f2, w_bf, cm):
    bf16 = jnp.bfloat16
    m = _leaky(f1 + jnp.max(f2))
    u_row = (f1 - m).astype(bf16)
    v_row = (ALPHA * f1 - m).astype(bf16)
    f2_bf = f2.astype(bf16)
    f2s = (ALPHA * f2).astype(bf16)
    x = jnp.maximum(u_row + f2_bf, v_row + f2s)
    p = adjv * jnp.exp2(x)
    hpe = jnp.dot(p, w_bf,
                  preferred_element_type=jnp.float32)
    hp = hpe[:, :D]
    s = hpe[:, D:D + 1]
    safe = jnp.where(s > 0, s, 1.0)
    return jnp.where(s > 0, hp / safe, cm)


def _with_ones_col(w_bf):
    rows = w_bf.shape[0]
    lane = jax.lax.broadcasted_iota(jnp.int32, (rows, D), 1)
    ones_blk = jnp.where(lane == 0, 1.0, 0.0).astype(jnp.bfloat16)
    return jnp.concatenate([w_bf, ones_blk], axis=1)


def _mega_body(x_ref, adj_ref, w1a_ref, w1b_ref, a1a_ref, a1b_ref,
               wout_ref, aout_ref, out_ref,
               wa_ref, wb_ref, f1a_ref, f2a_ref, f1b_ref, f2b_ref,
               cma_ref, cmb_ref,
               who_ref, f1o_ref, f2o_ref, cmo_ref):
    i = pl.program_id(0)

    @pl.when(i == 0)
    def _prewh():
        x = x_ref[...]
        wa = jnp.dot(x, w1a_ref[...], preferred_element_type=jnp.float32)
        wb = jnp.dot(x, w1b_ref[...], preferred_element_type=jnp.float32)
        wa_ref[...] = _with_ones_col(wa.astype(jnp.bfloat16))
        wb_ref[...] = _with_ones_col(wb.astype(jnp.bfloat16))
        f1a_ref[...], f2a_ref[...] = _logit_vecs(wa, a1a_ref[...])
        f1b_ref[...], f2b_ref[...] = _logit_vecs(wb, a1b_ref[...])
        cma_ref[...] = jnp.mean(wa, axis=0, keepdims=True)
        cmb_ref[...] = jnp.mean(wb, axis=0, keepdims=True)
        cmo_ref[...] = jnp.zeros_like(cmo_ref)

    @pl.when((i >= 1) & (i <= NBLK))
    def _layer1():
        r0 = (i - 1) * R
        adjv = adj_ref[...].astype(jnp.bfloat16)
        ha = _elu(_att_block(adjv, f1a_ref[pl.ds(r0, R), :], f2a_ref[...],
                             wa_ref[...], cma_ref[...]))
        hb = _elu(_att_block(adjv, f1b_ref[pl.ds(r0, R), :], f2b_ref[...],
                             wb_ref[...], cmb_ref[...]))
        h1 = jnp.concatenate([ha, hb], axis=1)
        who = jnp.dot(h1, wout_ref[...], preferred_element_type=jnp.float32)
        who_ref[pl.ds(r0, R), :] = _with_ones_col(who.astype(jnp.bfloat16))
        f1o, f2o = _logit_vecs(who, aout_ref[...])
        f1o_ref[pl.ds(r0, R), :] = f1o
        f2o_ref[:, pl.ds(r0, R)] = f2o
        cmo_ref[...] += jnp.sum(who, axis=0, keepdims=True) * (1.0 / N)

    @pl.when(i > NBLK)
    def _layer2():
        r0 = (i - NBLK - 1) * R
        out = _elu(_att_block(adj_ref[...].astype(jnp.bfloat16),
                              f1o_ref[pl.ds(r0, R), :],
                              f2o_ref[...], who_ref[...], cmo_ref[...]))
        m = jnp.max(out, axis=1, keepdims=True)
        lse = jnp.log(jnp.sum(jnp.exp(out - m), axis=1, keepdims=True))
        out_ref[...] = out - m - lse


def _full(shape):
    return pl.BlockSpec(shape, lambda i: (0, 0))


def kernel(idx, adj, table, W1a, a1a, W1b, a1b, Wout, aout):
    features = _sc_gather(table, idx)

    f32 = jnp.float32
    vmem = pltpu.VMEM
    out = pl.pallas_call(
        _mega_body,
        grid=(2 * NBLK + 1,),
        in_specs=[_full((N, D)),
                  pl.BlockSpec((R, N), lambda i: ((i + NBLK - 1) % NBLK, 0)),
                  _full((D, NHID)), _full((D, NHID)),
                  _full((1, 2 * NHID)), _full((1, 2 * NHID)),
                  _full((2 * NHID, NCLASS)), _full((1, 2 * NCLASS))],
        out_specs=pl.BlockSpec(
            (R, NCLASS), lambda i: (jnp.maximum(i - NBLK - 1, 0), 0)),
        out_shape=jax.ShapeDtypeStruct((N, NCLASS), f32),
        scratch_shapes=[
            vmem((N, NHID + 128), jnp.bfloat16),
            vmem((N, NHID + 128), jnp.bfloat16),
            vmem((N, 1), f32), vmem((1, N), f32),
            vmem((N, 1), f32), vmem((1, N), f32),
            vmem((1, NHID), f32), vmem((1, NHID), f32),
            vmem((N, NCLASS + 128), jnp.bfloat16),
            vmem((N, 1), f32), vmem((1, N), f32),
            vmem((1, NCLASS), f32),
        ],
    )(features, adj, W1a, W1b, a1a.reshape(1, -1), a1b.reshape(1, -1),
      Wout, aout.reshape(1, -1))

    return out
